# baseline (speedup 1.0000x reference)
import functools

import jax
import jax.numpy as jnp
from jax import lax
from jax.experimental import pallas as pl
from jax.experimental.pallas import tpu as pltpu
from jax.experimental.pallas import tpu_sc as plsc

_EXPERTS = 64
_BT = 4096
_CH = 256


def _router_body(hs_ref, w_ref, b_ref, packed_ref, p1_ref):
    logits = lax.dot_general(
        hs_ref[...], w_ref[...], (((1,), (1,)), ((), ())),
        preferred_element_type=jnp.float32,
    )
    logits = logits + b_ref[...]
    ex = lax.broadcasted_iota(jnp.int32, logits.shape, 1)
    m1 = jnp.max(logits, axis=1, keepdims=True)
    i1 = jnp.min(jnp.where(logits == m1, ex, _EXPERTS), axis=1, keepdims=True)
    masked = jnp.where(ex == i1, -jnp.inf, logits)
    m2 = jnp.max(masked, axis=1, keepdims=True)
    i2 = jnp.min(jnp.where(masked == m2, ex, _EXPERTS), axis=1, keepdims=True)
    e = jnp.exp(m2 - m1)
    p1 = 1.0 / (1.0 + e)
    packed_ref[...] = jnp.reshape(i1 * _EXPERTS + i2, (logits.shape[0],))
    p1_ref[...] = jnp.reshape(p1, (logits.shape[0],))


def _topk_compact(hidden_states, weight, bias):
    tokens, hidden = hidden_states.shape
    return pl.pallas_call(
        _router_body,
        grid=(tokens // _BT,),
        in_specs=[
            pl.BlockSpec((_BT, hidden), lambda i: (i, 0)),
            pl.BlockSpec((_EXPERTS, hidden), lambda i: (0, 0)),
            pl.BlockSpec((1, _EXPERTS), lambda i: (0, 0)),
        ],
        out_specs=[
            pl.BlockSpec((_BT,), lambda i: (i,)),
            pl.BlockSpec((_BT,), lambda i: (i,)),
        ],
        out_shape=[
            jax.ShapeDtypeStruct((tokens,), jnp.int32),
            jax.ShapeDtypeStruct((tokens,), jnp.float32),
        ],
    )(hidden_states, weight, bias.reshape(1, _EXPERTS))


def _make_finalize(tokens):
    info = plsc.get_sparse_core_info()
    nw = info.num_cores * info.num_subcores
    tpw = tokens // nw
    n_ch = tpw // _CH
    mesh = plsc.VectorSubcoreMesh(core_axis_name="c", subcore_axis_name="s")

    @functools.partial(
        pl.kernel,
        out_type=(
            jax.ShapeDtypeStruct((tokens, _EXPERTS), jnp.float32),
            jax.ShapeDtypeStruct((tokens, 2), jnp.int32),
        ),
        mesh=mesh,
        scratch_types=[
            pltpu.VMEM((tpw,), jnp.int32),
            pltpu.VMEM((tpw,), jnp.float32),
            pltpu.VMEM((_CH, _EXPERTS), jnp.float32),
            pltpu.VMEM((_CH, 2), jnp.int32),
        ],
        compiler_params=pltpu.CompilerParams(needs_layout_passes=False),
    )
    def finalize(packed_hbm, p1_hbm, scores_hbm, idx_hbm,
                 ip_v, p1_v, buf_v, ib_v):
        wid = lax.axis_index("s") * info.num_cores + lax.axis_index("c")
        base = wid * tpw
        pltpu.sync_copy(packed_hbm.at[pl.ds(base, tpw)], ip_v)
        pltpu.sync_copy(p1_hbm.at[pl.ds(base, tpw)], p1_v)

        zeros = jnp.zeros((16,), jnp.float32)
        lane = lax.iota(jnp.int32, 16)
        col0 = jnp.zeros((16,), jnp.int32)
        col1 = jnp.ones((16,), jnp.int32)

        for ch in range(n_ch):
            def zero_body(t, _):
                row = buf_v.at[t]
                for c in range(_EXPERTS // 16):
                    row[pl.ds(c * 16, 16)] = zeros
                return 0

            lax.fori_loop(0, _CH, zero_body, 0, unroll=4)

            def scat_body(i, _):
                tok = lane + i * 16
                ip = ip_v[pl.ds(ch * _CH + i * 16, 16)]
                p1 = p1_v[pl.ds(ch * _CH + i * 16, 16)]
                i1 = ip >> 6
                i2 = ip & 63
                plsc.store_scatter(buf_v, [tok, i1], p1)
                plsc.store_scatter(buf_v, [tok, i2], 1.0 - p1)
                plsc.store_scatter(ib_v, [tok, col0], i1)
                plsc.store_scatter(ib_v, [tok, col1], i2)
                return 0

            lax.fori_loop(0, _CH // 16, scat_body, 0, unroll=2)
            pltpu.sync_copy(
                buf_v, scores_hbm.at[pl.ds(base + ch * _CH, _CH), :])
            pltpu.sync_copy(
                ib_v, idx_hbm.at[pl.ds(base + ch * _CH, _CH), :])

    return finalize


def kernel(hidden_states, weight, bias):
    tokens, _ = hidden_states.shape
    packed, p1 = _topk_compact(hidden_states, weight, bias)
    scores, idx = _make_finalize(tokens)(packed, p1)
    return scores, idx


# speedup vs baseline: 1.0003x; 1.0003x over previous
import functools

import jax
import jax.numpy as jnp
from jax import lax
from jax.experimental import pallas as pl
from jax.experimental.pallas import tpu as pltpu
from jax.experimental.pallas import tpu_sc as plsc

_EXPERTS = 64
_BT = 4096
_CH = 256


def _router_body(hs_ref, w_ref, b_ref, packed_ref, p1_ref):
    logits = lax.dot_general(
        hs_ref[...], w_ref[...], (((1,), (1,)), ((), ())),
        preferred_element_type=jnp.float32,
    )
    logits = logits + b_ref[...]
    ex = lax.broadcasted_iota(jnp.int32, logits.shape, 1)
    m1 = jnp.max(logits, axis=1, keepdims=True)
    i1 = jnp.min(jnp.where(logits == m1, ex, _EXPERTS), axis=1, keepdims=True)
    masked = jnp.where(ex == i1, -jnp.inf, logits)
    m2 = jnp.max(masked, axis=1, keepdims=True)
    i2 = jnp.min(jnp.where(masked == m2, ex, _EXPERTS), axis=1, keepdims=True)
    e = jnp.exp(m2 - m1)
    p1 = 1.0 / (1.0 + e)
    packed_ref[...] = jnp.reshape(i1 * _EXPERTS + i2, (logits.shape[0],))
    p1_ref[...] = jnp.reshape(p1, (logits.shape[0],))


def _topk_compact(hidden_states, weight, bias):
    tokens, hidden = hidden_states.shape
    return pl.pallas_call(
        _router_body,
        grid=(tokens // _BT,),
        in_specs=[
            pl.BlockSpec((_BT, hidden), lambda i: (i, 0)),
            pl.BlockSpec((_EXPERTS, hidden), lambda i: (0, 0)),
            pl.BlockSpec((1, _EXPERTS), lambda i: (0, 0)),
        ],
        out_specs=[
            pl.BlockSpec((_BT,), lambda i: (i,)),
            pl.BlockSpec((_BT,), lambda i: (i,)),
        ],
        out_shape=[
            jax.ShapeDtypeStruct((tokens,), jnp.int32),
            jax.ShapeDtypeStruct((tokens,), jnp.float32),
        ],
    )(hidden_states, weight, bias.reshape(1, _EXPERTS))


def _make_finalize(tokens):
    info = plsc.get_sparse_core_info()
    nw = info.num_cores * info.num_subcores
    tpw = tokens // nw
    n_ch = tpw // _CH
    mesh = plsc.VectorSubcoreMesh(core_axis_name="c", subcore_axis_name="s")

    @functools.partial(
        pl.kernel,
        out_type=(
            jax.ShapeDtypeStruct((tokens, _EXPERTS), jnp.float32),
            jax.ShapeDtypeStruct((tokens, 2), jnp.int32),
        ),
        mesh=mesh,
        scratch_types=[
            pltpu.VMEM((tpw,), jnp.int32),
            pltpu.VMEM((tpw,), jnp.float32),
            pltpu.VMEM((_CH, _EXPERTS), jnp.float32),
            pltpu.VMEM((_CH, 2), jnp.int32),
        ],
        compiler_params=pltpu.CompilerParams(
            needs_layout_passes=False, skip_device_barrier=True),
    )
    def finalize(packed_hbm, p1_hbm, scores_hbm, idx_hbm,
                 ip_v, p1_v, buf_v, ib_v):
        wid = lax.axis_index("s") * info.num_cores + lax.axis_index("c")
        base = wid * tpw
        pltpu.sync_copy(packed_hbm.at[pl.ds(base, tpw)], ip_v)
        pltpu.sync_copy(p1_hbm.at[pl.ds(base, tpw)], p1_v)

        zeros = jnp.zeros((16,), jnp.float32)
        lane = lax.iota(jnp.int32, 16)
        col0 = jnp.zeros((16,), jnp.int32)
        col1 = jnp.ones((16,), jnp.int32)

        for ch in range(n_ch):
            def zero_body(t, _):
                row = buf_v.at[t]
                for c in range(_EXPERTS // 16):
                    row[pl.ds(c * 16, 16)] = zeros
                return 0

            lax.fori_loop(0, _CH, zero_body, 0, unroll=4)

            def scat_body(i, _):
                tok = lane + i * 16
                ip = ip_v[pl.ds(ch * _CH + i * 16, 16)]
                p1 = p1_v[pl.ds(ch * _CH + i * 16, 16)]
                i1 = ip >> 6
                i2 = ip & 63
                plsc.store_scatter(buf_v, [tok, i1], p1)
                plsc.store_scatter(buf_v, [tok, i2], 1.0 - p1)
                plsc.store_scatter(ib_v, [tok, col0], i1)
                plsc.store_scatter(ib_v, [tok, col1], i2)
                return 0

            lax.fori_loop(0, _CH // 16, scat_body, 0, unroll=2)
            pltpu.sync_copy(
                buf_v, scores_hbm.at[pl.ds(base + ch * _CH, _CH), :])
            pltpu.sync_copy(
                ib_v, idx_hbm.at[pl.ds(base + ch * _CH, _CH), :])

    return finalize


def kernel(hidden_states, weight, bias):
    tokens, _ = hidden_states.shape
    packed, p1 = _topk_compact(hidden_states, weight, bias)
    scores, idx = _make_finalize(tokens)(packed, p1)
    return scores, idx


# speedup vs baseline: 1.7138x; 1.7132x over previous
import jax
import jax.numpy as jnp
from jax import lax
from jax.experimental import pallas as pl

_EXPERTS = 64
_BT = 4096


def _router_body(hs_ref, w_ref, b_ref, scores_ref, packed_ref):
    logits = lax.dot_general(
        hs_ref[...], w_ref[...], (((1,), (1,)), ((), ())),
        preferred_element_type=jnp.float32,
    )
    logits = logits + b_ref[...]
    ex = lax.broadcasted_iota(jnp.int32, logits.shape, 1)
    m1 = jnp.max(logits, axis=1, keepdims=True)
    i1 = jnp.min(jnp.where(logits == m1, ex, _EXPERTS), axis=1, keepdims=True)
    masked = jnp.where(ex == i1, -jnp.inf, logits)
    m2 = jnp.max(masked, axis=1, keepdims=True)
    i2 = jnp.min(jnp.where(masked == m2, ex, _EXPERTS), axis=1, keepdims=True)
    e = jnp.exp(m2 - m1)
    p1 = 1.0 / (1.0 + e)
    p2 = e / (1.0 + e)
    scores_ref[...] = jnp.where(ex == i1, p1, jnp.where(ex == i2, p2, 0.0))
    packed_ref[...] = jnp.reshape(i1 * _EXPERTS + i2, (logits.shape[0],))


def kernel(hidden_states, weight, bias):
    tokens, hidden = hidden_states.shape
    scores, packed = pl.pallas_call(
        _router_body,
        grid=(tokens // _BT,),
        in_specs=[
            pl.BlockSpec((_BT, hidden), lambda i: (i, 0)),
            pl.BlockSpec((_EXPERTS, hidden), lambda i: (0, 0)),
            pl.BlockSpec((1, _EXPERTS), lambda i: (0, 0)),
        ],
        out_specs=[
            pl.BlockSpec((_BT, _EXPERTS), lambda i: (i, 0)),
            pl.BlockSpec((_BT,), lambda i: (i,)),
        ],
        out_shape=[
            jax.ShapeDtypeStruct((tokens, _EXPERTS), jnp.float32),
            jax.ShapeDtypeStruct((tokens,), jnp.int32),
        ],
    )(hidden_states, weight, bias.reshape(1, _EXPERTS))
    idx = jnp.stack([packed >> 6, packed & (_EXPERTS - 1)], axis=-1)
    return scores, idx
